# baseline (device time: 46055 ns/iter reference)
import jax
import jax.numpy as jnp
from jax import lax
from jax.experimental import pallas as pl
from jax.experimental.pallas import tpu as pltpu

N_DEV = 4


def kernel(x, w_mat):
    m_global, k_per = x.shape
    _, n = w_mat.shape
    m_per = m_global // N_DEV

    def body(x_ref, w_ref, out_ref, comm_ref, send_sems, recv_sems):
        p = lax.axis_index("i")
        left = lax.rem(p + N_DEV - 1, N_DEV)
        right = lax.rem(p + 1, N_DEV)

        barrier_sem = pltpu.get_barrier_semaphore()
        for nbr in [left, right]:
            pl.semaphore_signal(
                barrier_sem, inc=1,
                device_id=(nbr,), device_id_type=pl.DeviceIdType.MESH,
            )
        pl.semaphore_wait(barrier_sem, 2)

        def partial_chunk(c):
            return jnp.dot(
                x_ref[pl.ds(c * m_per, m_per), :], w_ref[:, :],
                preferred_element_type=jnp.float32,
            )

        c0 = lax.rem(p + N_DEV - 1, N_DEV)
        comm_ref[0, :, :] = partial_chunk(c0)

        for t in range(N_DEV - 1):
            send_slot = t % 2
            recv_slot = (t + 1) % 2
            rdma = pltpu.make_async_remote_copy(
                src_ref=comm_ref.at[send_slot],
                dst_ref=comm_ref.at[recv_slot],
                send_sem=send_sems.at[t],
                recv_sem=recv_sems.at[t],
                device_id=(right,),
                device_id_type=pl.DeviceIdType.MESH,
            )
            rdma.start()
            c = lax.rem(p + N_DEV - 2 - t + N_DEV, N_DEV)
            partial = partial_chunk(c)
            rdma.wait()
            if t < N_DEV - 2:
                comm_ref[recv_slot, :, :] = comm_ref[recv_slot, :, :] + partial
            else:
                out_ref[:, :] = comm_ref[recv_slot, :, :] + partial

    return pl.pallas_call(
        body,
        out_shape=jax.ShapeDtypeStruct((m_per, n), jnp.float32),
        in_specs=[
            pl.BlockSpec(memory_space=pltpu.VMEM),
            pl.BlockSpec(memory_space=pltpu.VMEM),
        ],
        out_specs=pl.BlockSpec(memory_space=pltpu.VMEM),
        scratch_shapes=[
            pltpu.VMEM((2, m_per, n), jnp.float32),
            pltpu.SemaphoreType.DMA((N_DEV - 1,)),
            pltpu.SemaphoreType.DMA((N_DEV - 1,)),
        ],
        compiler_params=pltpu.CompilerParams(collective_id=0),
    )(x, w_mat)


# device time: 29274 ns/iter; 1.5732x vs baseline; 1.5732x over previous
import jax
import jax.numpy as jnp
from jax import lax
from jax.experimental import pallas as pl
from jax.experimental.pallas import tpu as pltpu

N_DEV = 4


def kernel(x, w_mat):
    m_global, k_per = x.shape
    _, n = w_mat.shape
    m_per = m_global // N_DEV
    nh = n // 2

    def body(x_ref, w_ref, out_ref, comm_r, comm_l,
             send_sems_r, recv_sems_r, send_sems_l, recv_sems_l):
        p = lax.axis_index("i")
        left = lax.rem(p + N_DEV - 1, N_DEV)
        right = lax.rem(p + 1, N_DEV)

        barrier_sem = pltpu.get_barrier_semaphore()
        for nbr in [left, right]:
            pl.semaphore_signal(
                barrier_sem, inc=1,
                device_id=(nbr,), device_id_type=pl.DeviceIdType.MESH,
            )
        pl.semaphore_wait(barrier_sem, 2)

        def partial_half(c, lo):
            return jnp.dot(
                x_ref[pl.ds(c * m_per, m_per), :], w_ref[:, pl.ds(lo, nh)],
                preferred_element_type=jnp.float32,
            )

        comm_r[0, :, :] = partial_half(lax.rem(p + N_DEV - 1, N_DEV), 0)
        comm_l[0, :, :] = partial_half(lax.rem(p + 1, N_DEV), nh)

        for t in range(N_DEV - 1):
            send_slot = t % 2
            recv_slot = (t + 1) % 2
            rdma_r = pltpu.make_async_remote_copy(
                src_ref=comm_r.at[send_slot],
                dst_ref=comm_r.at[recv_slot],
                send_sem=send_sems_r.at[t],
                recv_sem=recv_sems_r.at[t],
                device_id=(right,),
                device_id_type=pl.DeviceIdType.MESH,
            )
            rdma_l = pltpu.make_async_remote_copy(
                src_ref=comm_l.at[send_slot],
                dst_ref=comm_l.at[recv_slot],
                send_sem=send_sems_l.at[t],
                recv_sem=recv_sems_l.at[t],
                device_id=(left,),
                device_id_type=pl.DeviceIdType.MESH,
            )
            rdma_r.start()
            rdma_l.start()
            c_r = lax.rem(p + N_DEV - 2 - t + N_DEV, N_DEV)
            c_l = lax.rem(p + 2 + t, N_DEV)
            part_r = partial_half(c_r, 0)
            part_l = partial_half(c_l, nh)
            rdma_r.wait()
            rdma_l.wait()
            if t < N_DEV - 2:
                comm_r[recv_slot, :, :] = comm_r[recv_slot, :, :] + part_r
                comm_l[recv_slot, :, :] = comm_l[recv_slot, :, :] + part_l
            else:
                out_ref[:, pl.ds(0, nh)] = comm_r[recv_slot, :, :] + part_r
                out_ref[:, pl.ds(nh, nh)] = comm_l[recv_slot, :, :] + part_l

    return pl.pallas_call(
        body,
        out_shape=jax.ShapeDtypeStruct((m_per, n), jnp.float32),
        in_specs=[
            pl.BlockSpec(memory_space=pltpu.VMEM),
            pl.BlockSpec(memory_space=pltpu.VMEM),
        ],
        out_specs=pl.BlockSpec(memory_space=pltpu.VMEM),
        scratch_shapes=[
            pltpu.VMEM((2, m_per, nh), jnp.float32),
            pltpu.VMEM((2, m_per, nh), jnp.float32),
            pltpu.SemaphoreType.DMA((N_DEV - 1,)),
            pltpu.SemaphoreType.DMA((N_DEV - 1,)),
            pltpu.SemaphoreType.DMA((N_DEV - 1,)),
            pltpu.SemaphoreType.DMA((N_DEV - 1,)),
        ],
        compiler_params=pltpu.CompilerParams(collective_id=0),
    )(x, w_mat)


# device time: 26363 ns/iter; 1.7470x vs baseline; 1.1104x over previous
import functools

import jax
import jax.numpy as jnp
from jax import lax
from jax.experimental import pallas as pl
from jax.experimental.pallas import tpu as pltpu

N_DEV = 4
N_HOP = N_DEV - 1
S = 2


def kernel(x, w_mat):
    m_global, k_per = x.shape
    _, n = w_mat.shape
    m_per = m_global // N_DEV
    nh = n // 2
    ns = nh // S

    def body(x_ref, w_ref, out_ref, *scratch):
        comm = scratch[0:4]
        ssems = scratch[4:8]
        rsems = scratch[8:12]

        p = lax.axis_index("i")
        left = lax.rem(p + N_DEV - 1, N_DEV)
        right = lax.rem(p + 1, N_DEV)
        targets = [right, right, left, left]

        barrier_sem = pltpu.get_barrier_semaphore()
        for nbr in [left, right]:
            pl.semaphore_signal(
                barrier_sem, inc=1,
                device_id=(nbr,), device_id_type=pl.DeviceIdType.MESH,
            )
        pl.semaphore_wait(barrier_sem, 2)

        rdmas = {}
        for t in range(N_HOP):
            for k in range(4):
                rdmas[(t, k)] = pltpu.make_async_remote_copy(
                    src_ref=comm[k].at[t],
                    dst_ref=comm[k].at[t + 1],
                    send_sem=ssems[k].at[t],
                    recv_sem=rsems[k].at[t],
                    device_id=(targets[k],),
                    device_id_type=pl.DeviceIdType.MESH,
                )

        def partial_r(c):
            return jnp.dot(
                x_ref[pl.ds(c * m_per, m_per), :], w_ref[:, pl.ds(0, nh)],
                preferred_element_type=jnp.float32,
            )

        def partial_l(c):
            return jnp.dot(
                x_ref[pl.ds(c * m_per, m_per), :], w_ref[:, pl.ds(nh, nh)],
                preferred_element_type=jnp.float32,
            )

        seed_r = partial_r(lax.rem(p + N_DEV - 1, N_DEV))
        comm[0][0, :, :] = seed_r[:, 0:ns]
        rdmas[(0, 0)].start()
        comm[1][0, :, :] = seed_r[:, ns:2 * ns]
        rdmas[(0, 1)].start()
        seed_l = partial_l(lax.rem(p + 1, N_DEV))
        comm[2][0, :, :] = seed_l[:, 0:ns]
        rdmas[(0, 2)].start()
        comm[3][0, :, :] = seed_l[:, ns:2 * ns]
        rdmas[(0, 3)].start()

        for t in range(N_HOP):
            part_r = partial_r(lax.rem(p + 2 * N_DEV - 2 - t, N_DEV))
            part_l = partial_l(lax.rem(p + 2 + t, N_DEV))
            halves = [
                (0, part_r[:, 0:ns], 0),
                (2, part_l[:, 0:ns], nh),
                (1, part_r[:, ns:2 * ns], ns),
                (3, part_l[:, ns:2 * ns], nh + ns),
            ]
            for k, part, out_lo in halves:
                rdmas[(t, k)].wait_recv()
                acc = comm[k][t + 1, :, :] + part
                if t < N_HOP - 1:
                    comm[k][t + 1, :, :] = acc
                    rdmas[(t + 1, k)].start()
                else:
                    out_ref[:, pl.ds(out_lo, ns)] = acc

        for t in range(N_HOP):
            for k in range(4):
                rdmas[(t, k)].wait_send()

        @functools.partial(
            pl.run_scoped, second_barrier=pltpu.SemaphoreType.REGULAR
        )
        def _(second_barrier):
            for nbr in [left, right]:
                pl.semaphore_signal(
                    second_barrier, inc=1,
                    device_id=(nbr,), device_id_type=pl.DeviceIdType.MESH,
                )
            pl.semaphore_wait(second_barrier, 2)

    return pl.pallas_call(
        body,
        out_shape=jax.ShapeDtypeStruct((m_per, n), jnp.float32),
        in_specs=[
            pl.BlockSpec(memory_space=pltpu.VMEM),
            pl.BlockSpec(memory_space=pltpu.VMEM),
        ],
        out_specs=pl.BlockSpec(memory_space=pltpu.VMEM),
        scratch_shapes=(
            [pltpu.VMEM((N_HOP + 1, m_per, ns), jnp.float32)] * 4
            + [pltpu.SemaphoreType.DMA((N_HOP,))] * 8
        ),
        compiler_params=pltpu.CompilerParams(collective_id=0),
    )(x, w_mat)


# device time: 25834 ns/iter; 1.7827x vs baseline; 1.0205x over previous
import functools

import jax
import jax.numpy as jnp
from jax import lax
from jax.experimental import pallas as pl
from jax.experimental.pallas import tpu as pltpu

N_DEV = 4
N_HOP = N_DEV - 1
S = 2


def kernel(x, w_mat):
    m_global, k_per = x.shape
    _, n = w_mat.shape
    m_per = m_global // N_DEV
    nh = n // 2
    ns = nh // S

    def body(x_ref, w_ref, out_ref, *scratch):
        comm = scratch[0:4]
        ssems = scratch[4:8]
        rsems = scratch[8:12]

        p = lax.axis_index("i")
        left = lax.rem(p + N_DEV - 1, N_DEV)
        right = lax.rem(p + 1, N_DEV)
        targets = [right, right, left, left]

        barrier_sem = pltpu.get_barrier_semaphore()
        for nbr in [left, right]:
            pl.semaphore_signal(
                barrier_sem, inc=1,
                device_id=(nbr,), device_id_type=pl.DeviceIdType.MESH,
            )

        rdmas = {}
        for t in range(N_HOP):
            for k in range(4):
                rdmas[(t, k)] = pltpu.make_async_remote_copy(
                    src_ref=comm[k].at[t],
                    dst_ref=comm[k].at[t + 1],
                    send_sem=ssems[k].at[t],
                    recv_sem=rsems[k].at[t],
                    device_id=(targets[k],),
                    device_id_type=pl.DeviceIdType.MESH,
                )

        def partial_r(c):
            return jnp.dot(
                x_ref[pl.ds(c * m_per, m_per), :], w_ref[:, pl.ds(0, nh)],
                preferred_element_type=jnp.float32,
            )

        def partial_l(c):
            return jnp.dot(
                x_ref[pl.ds(c * m_per, m_per), :], w_ref[:, pl.ds(nh, nh)],
                preferred_element_type=jnp.float32,
            )

        seed_r = partial_r(lax.rem(p + N_DEV - 1, N_DEV))
        comm[0][0, :, :] = seed_r[:, 0:ns]
        comm[1][0, :, :] = seed_r[:, ns:2 * ns]
        seed_l = partial_l(lax.rem(p + 1, N_DEV))
        comm[2][0, :, :] = seed_l[:, 0:ns]
        comm[3][0, :, :] = seed_l[:, ns:2 * ns]

        pl.semaphore_wait(barrier_sem, 2)
        for k in range(4):
            rdmas[(0, k)].start()

        for t in range(N_HOP):
            part_r = partial_r(lax.rem(p + 2 * N_DEV - 2 - t, N_DEV))
            part_l = partial_l(lax.rem(p + 2 + t, N_DEV))
            halves = [
                (0, part_r[:, 0:ns], 0),
                (2, part_l[:, 0:ns], nh),
                (1, part_r[:, ns:2 * ns], ns),
                (3, part_l[:, ns:2 * ns], nh + ns),
            ]
            for k, part, out_lo in halves:
                rdmas[(t, k)].wait_recv()
                acc = comm[k][t + 1, :, :] + part
                if t < N_HOP - 1:
                    comm[k][t + 1, :, :] = acc
                    rdmas[(t + 1, k)].start()
                else:
                    out_ref[:, pl.ds(out_lo, ns)] = acc

        for t in range(N_HOP):
            for k in range(4):
                rdmas[(t, k)].wait_send()

        @functools.partial(
            pl.run_scoped, second_barrier=pltpu.SemaphoreType.REGULAR
        )
        def _(second_barrier):
            for nbr in [left, right]:
                pl.semaphore_signal(
                    second_barrier, inc=1,
                    device_id=(nbr,), device_id_type=pl.DeviceIdType.MESH,
                )
            pl.semaphore_wait(second_barrier, 2)

    return pl.pallas_call(
        body,
        out_shape=jax.ShapeDtypeStruct((m_per, n), jnp.float32),
        in_specs=[
            pl.BlockSpec(memory_space=pltpu.VMEM),
            pl.BlockSpec(memory_space=pltpu.VMEM),
        ],
        out_specs=pl.BlockSpec(memory_space=pltpu.VMEM),
        scratch_shapes=(
            [pltpu.VMEM((N_HOP + 1, m_per, ns), jnp.float32)] * 4
            + [pltpu.SemaphoreType.DMA((N_HOP,))] * 8
        ),
        compiler_params=pltpu.CompilerParams(collective_id=0),
    )(x, w_mat)
